# baseline (speedup 1.0000x reference)
import functools

import jax
import jax.numpy as jnp
from jax import lax
from jax.experimental import pallas as pl
from jax.experimental.pallas import tpu as pltpu
from jax.experimental.pallas import tpu_sc as plsc

_NUM_CORES = 2
_NUM_SUBCORES = 16
_NW = _NUM_CORES * _NUM_SUBCORES

_NBUF = 4
_BLK = 128


def _make_gather(num_rows, dim, batch_sz, nfields):
    npairs = num_rows // 2
    pdim = 2 * dim
    nbg = batch_sz // _BLK
    nblocks = nfields * nbg
    blocks_per_w = nblocks // _NW
    idx_per_w = blocks_per_w * _BLK
    ndg = dim // 8
    mesh = plsc.VectorSubcoreMesh(core_axis_name="c", subcore_axis_name="s")

    @functools.partial(
        pl.kernel,
        mesh=mesh,
        compiler_params=pltpu.CompilerParams(
            use_tc_tiling_on_sc=False, needs_layout_passes=False
        ),
        out_type=jax.ShapeDtypeStruct((nfields * ndg * nbg * 8 * _BLK,), jnp.float32),
        scratch_types=[
            pltpu.VMEM((idx_per_w,), jnp.int32),
            pltpu.VMEM((_NBUF, _BLK, pdim), jnp.float32),
            pltpu.VMEM((2, dim * _BLK), jnp.float32),
            pltpu.SemaphoreType.DMA((_NBUF,)),
            pltpu.SemaphoreType.DMA((2,)),
        ],
    )
    def gather(table_hbm, idx_hbm, out_hbm, idx_v, rows_v, obuf_v,
               gsem, ssem):
        wid = lax.axis_index("s") * _NUM_CORES + lax.axis_index("c")
        base = wid * idx_per_w
        pltpu.sync_copy(idx_hbm.at[pl.ds(base, idx_per_w)], idx_v)

        def start_gather(i, b):
            pltpu.async_copy(
                table_hbm.at[idx_v.at[pl.ds(i * _BLK, _BLK)]],
                rows_v.at[b],
                gsem.at[b],
            )

        def wait_gather(i, b):
            pltpu.make_async_copy(
                table_hbm.at[idx_v.at[pl.ds(i * _BLK, _BLK)]],
                rows_v.at[b],
                gsem.at[b],
            ).wait()

        def store_block(i, o):
            blk = base // _BLK + i
            f = lax.div(blk, nbg)
            bg = lax.rem(blk, nbg)
            for dg in range(ndg):
                pltpu.async_copy(
                    obuf_v.at[o, pl.ds(dg * 8 * _BLK, 8 * _BLK)],
                    out_hbm.at[pl.ds(((f * ndg + dg) * nbg + bg) * 8 * _BLK,
                                     8 * _BLK)],
                    ssem.at[o],
                )

        def wait_stores(i, o):
            for dg in range(ndg):
                pltpu.make_async_copy(
                    obuf_v.at[o, pl.ds(dg * 8 * _BLK, 8 * _BLK)],
                    out_hbm.at[pl.ds(0, 8 * _BLK)],
                    ssem.at[o],
                ).wait()

        lanes = lax.broadcasted_iota(jnp.int32, (16,), 0)
        kvecs = [m * 16 + lanes for m in range(_BLK // 16)]

        def transpose_block(i, b, o):
            @pl.loop(0, dim, unroll=4)
            def _(d):
                col = lanes * 0 + d
                for m in range(_BLK // 16):
                    x = plsc.load_gather(rows_v.at[b], [kvecs[m], col])
                    obuf_v[o, pl.ds(d * _BLK + m * 16, 16)] = x

        for i in range(_NBUF):
            start_gather(i, i)

        def body(i, b):
            o = b & 1
            wait_stores(i - 2, o)
            wait_gather(i, b)
            transpose_block(i, b, o)
            store_block(i, o)
            start_gather(i + _NBUF, b)

        for i in range(2):
            b = i % _NBUF
            o = b & 1
            wait_gather(i, b)
            transpose_block(i, b, o)
            store_block(i, o)
            start_gather(i + _NBUF, b)

        n_steady = (blocks_per_w - 2 - _NBUF) // _NBUF

        @pl.loop(0, n_steady)
        def _(j):
            for bb in range(_NBUF):
                body(2 + j * _NBUF + bb, (2 + bb) % _NBUF)

        for i in range(2 + n_steady * _NBUF, blocks_per_w - _NBUF):
            body(i, i % _NBUF)
        for i in range(blocks_per_w - _NBUF, blocks_per_w):
            b = i % _NBUF
            o = b & 1
            wait_stores(i - 2, o)
            wait_gather(i, b)
            transpose_block(i, b, o)
            store_block(i, o)
        for i in range(blocks_per_w - 2, blocks_per_w):
            wait_stores(i, i % 2)

    return gather


def kernel(indices, weight):
    num_rows, dim = weight.shape
    nbatch, nfields = indices.shape
    wpad = jnp.pad(weight, ((0, 0), (0, dim)))
    flat_t = indices.T.reshape(-1)
    gather = _make_gather(num_rows, dim, nbatch, nfields)
    o1 = gather(wpad, flat_t)
    o5 = o1.reshape(nfields, dim // 8, nbatch // _BLK, 8, _BLK)
    return o5.transpose(2, 4, 0, 1, 3).reshape(nbatch, nfields, dim)


# speedup vs baseline: 1.3259x; 1.3259x over previous
import functools

import jax
import jax.numpy as jnp
from jax import lax
from jax.experimental import pallas as pl
from jax.experimental.pallas import tpu as pltpu
from jax.experimental.pallas import tpu_sc as plsc

_NUM_CORES = 2
_NUM_SUBCORES = 16
_NW = _NUM_CORES * _NUM_SUBCORES

_NBUF = 4
_CHUNK = 208


def _make_gather(num_rows, dim, batch, nfields):
    assert batch % (_NW * _CHUNK) == 0
    b_per_w = batch // _NW
    nchunks = b_per_w // _CHUNK
    assert nchunks >= 2 * _NBUF
    pdim = 2 * dim
    mesh = plsc.VectorSubcoreMesh(core_axis_name="c", subcore_axis_name="s")

    @functools.partial(
        pl.kernel,
        mesh=mesh,
        compiler_params=pltpu.CompilerParams(use_tc_tiling_on_sc=False),
        out_type=jax.ShapeDtypeStruct((batch, dim), jnp.float32),
        scratch_types=[
            pltpu.VMEM((b_per_w,), jnp.int32),
            pltpu.VMEM((_NBUF, _CHUNK, pdim), jnp.float32),
            pltpu.SemaphoreType.DMA((_NBUF,)),
            pltpu.SemaphoreType.DMA((_NBUF,)),
        ],
    )
    def gather(table_hbm, idx_hbm, out_hbm, idx_v, rows_v, gsem, ssem):
        wid = lax.axis_index("s") * _NUM_CORES + lax.axis_index("c")
        base = wid * b_per_w
        pltpu.sync_copy(idx_hbm.at[pl.ds(base, b_per_w)], idx_v)
        out_flat = out_hbm

        def start_gather(g, b):
            pltpu.async_copy(
                table_hbm.at[idx_v.at[pl.ds(g * _CHUNK, _CHUNK)]],
                rows_v.at[b],
                gsem.at[b],
            )

        def start_store(g, b):
            pltpu.async_copy(
                rows_v.at[b, :, pl.ds(0, dim)],
                out_flat.at[pl.ds(base + g * _CHUNK, _CHUNK)],
                ssem.at[b],
            )

        def wait_gather(g, b):
            pltpu.make_async_copy(
                table_hbm.at[idx_v.at[pl.ds(g * _CHUNK, _CHUNK)]],
                rows_v.at[b],
                gsem.at[b],
            ).wait()

        def wait_store(g, b):
            pltpu.make_async_copy(
                rows_v.at[b, :, pl.ds(0, dim)],
                out_flat.at[pl.ds(base + g * _CHUNK, _CHUNK)],
                ssem.at[b],
            ).wait()

        for i in range(_NBUF):
            start_gather(i, i)
            if i == _NBUF - 1:
                wait_gather(0, 0)
                start_store(0, 0)

        @pl.loop(_NBUF, nchunks)
        def _(i):
            b = lax.rem(i, _NBUF)
            bp = lax.rem(i + 1, _NBUF)
            wait_store(i - _NBUF, b)
            start_gather(i, b)
            wait_gather(i - _NBUF + 1, bp)
            start_store(i - _NBUF + 1, bp)

        for i in range(nchunks, nchunks + _NBUF):
            b = i % _NBUF
            bp = (i + 1) % _NBUF
            wait_store(i - _NBUF, b)
            if i < nchunks + _NBUF - 1:
                wait_gather(i - _NBUF + 1, bp)
                start_store(i - _NBUF + 1, bp)

    return gather


def kernel(indices, weight):
    num_rows, dim = weight.shape
    nbatch, nfields = indices.shape
    flat_t = indices.T.reshape(-1)
    wpad = jnp.pad(weight, ((0, 0), (0, dim)))
    gather = _make_gather(num_rows, dim, flat_t.shape[0], nfields)
    out = gather(wpad, flat_t)
    return out.reshape(nfields, nbatch, dim).transpose(1, 0, 2)


# speedup vs baseline: 1.3853x; 1.0448x over previous
import functools

import jax
import jax.numpy as jnp
from jax import lax
from jax.experimental import pallas as pl
from jax.experimental.pallas import tpu as pltpu
from jax.experimental.pallas import tpu_sc as plsc

_NUM_CORES = 2
_NUM_SUBCORES = 16
_NW = _NUM_CORES * _NUM_SUBCORES

_NBUF = 4
_CHUNK = 208


def _make_gather(num_rows, dim, batch, nfields):
    assert batch % (_NW * _CHUNK) == 0
    b_per_w = batch // _NW
    nchunks = b_per_w // _CHUNK
    assert nchunks >= 2 * _NBUF
    mesh = plsc.VectorSubcoreMesh(core_axis_name="c", subcore_axis_name="s")

    @functools.partial(
        pl.kernel,
        mesh=mesh,
        compiler_params=pltpu.CompilerParams(use_tc_tiling_on_sc=False),
        out_type=jax.ShapeDtypeStruct((batch, dim), jnp.float32),
        scratch_types=[
            pltpu.VMEM((b_per_w,), jnp.int32),
            pltpu.VMEM((b_per_w,), jnp.int32),
            pltpu.VMEM((_NBUF, _CHUNK, dim), jnp.float32),
            pltpu.SemaphoreType.DMA((_NBUF,)),
            pltpu.SemaphoreType.DMA((_NBUF,)),
        ],
    )
    def gather(table_hbm, idx_hbm, out_hbm, idx_v, pidx_v, rows_v, gsem, ssem):
        wid = lax.axis_index("s") * _NUM_CORES + lax.axis_index("c")
        base = wid * b_per_w
        pltpu.sync_copy(idx_hbm.at[pl.ds(base, b_per_w)], idx_v)
        out_flat = out_hbm

        @pl.loop(0, b_per_w // 16)
        def _(g):
            v = idx_v[pl.ds(g * 16, 16)]
            pidx_v[pl.ds(g * 16, 16)] = v + v

        def start_gather(g, b):
            pltpu.async_copy(
                table_hbm.at[pidx_v.at[pl.ds(g * _CHUNK, _CHUNK)]],
                rows_v.at[b],
                gsem.at[b],
            )

        def start_store(g, b):
            pltpu.async_copy(
                rows_v.at[b],
                out_flat.at[pl.ds(base + g * _CHUNK, _CHUNK)],
                ssem.at[b],
            )

        def wait_gather(g, b):
            pltpu.make_async_copy(
                table_hbm.at[pidx_v.at[pl.ds(g * _CHUNK, _CHUNK)]],
                rows_v.at[b],
                gsem.at[b],
            ).wait()

        def wait_store(g, b):
            pltpu.make_async_copy(
                rows_v.at[b],
                out_flat.at[pl.ds(base + g * _CHUNK, _CHUNK)],
                ssem.at[b],
            ).wait()

        for i in range(_NBUF):
            start_gather(i, i)
            if i == _NBUF - 1:
                wait_gather(0, 0)
                start_store(0, 0)

        @pl.loop(_NBUF, nchunks)
        def _(i):
            b = lax.rem(i, _NBUF)
            bp = lax.rem(i + 1, _NBUF)
            wait_store(i - _NBUF, b)
            start_gather(i, b)
            wait_gather(i - _NBUF + 1, bp)
            start_store(i - _NBUF + 1, bp)

        for i in range(nchunks, nchunks + _NBUF):
            b = i % _NBUF
            bp = (i + 1) % _NBUF
            wait_store(i - _NBUF, b)
            if i < nchunks + _NBUF - 1:
                wait_gather(i - _NBUF + 1, bp)
                start_store(i - _NBUF + 1, bp)

    return gather


def kernel(indices, weight):
    num_rows, dim = weight.shape
    nbatch, nfields = indices.shape
    flat_t = indices.T.reshape(-1)
    wpad = jnp.pad(weight, ((0, 0), (0, dim))).reshape(2 * num_rows, dim)
    gather = _make_gather(num_rows, dim, flat_t.shape[0], nfields)
    out = gather(wpad, flat_t)
    return out.reshape(nfields, nbatch, dim).transpose(1, 0, 2)


# speedup vs baseline: 1.5417x; 1.1129x over previous
import functools

import jax
import jax.numpy as jnp
from jax import lax
from jax.experimental import pallas as pl
from jax.experimental.pallas import tpu as pltpu
from jax.experimental.pallas import tpu_sc as plsc

_NUM_CORES = 2
_NUM_SUBCORES = 16
_NW = _NUM_CORES * _NUM_SUBCORES

_NBUF = 4
_CHUNK = 208


def _make_gather(num_rows, dim, batch, nfields):
    assert batch % (_NW * _CHUNK) == 0
    b_per_w = batch // _NW
    nchunks = b_per_w // _CHUNK
    assert nchunks >= 2 * _NBUF
    mesh = plsc.VectorSubcoreMesh(core_axis_name="c", subcore_axis_name="s")

    @functools.partial(
        pl.kernel,
        mesh=mesh,
        compiler_params=pltpu.CompilerParams(use_tc_tiling_on_sc=False),
        out_type=jax.ShapeDtypeStruct((batch, 2 * dim), jnp.float32),
        scratch_types=[
            pltpu.VMEM((b_per_w,), jnp.int32),
            pltpu.VMEM((_NBUF, _CHUNK, 2 * dim), jnp.float32),
            pltpu.SemaphoreType.DMA((_NBUF,)),
            pltpu.SemaphoreType.DMA((_NBUF,)),
        ],
    )
    def gather(table_hbm, idx_hbm, out_hbm, idx_v, rows_v, gsem, ssem):
        wid = lax.axis_index("s") * _NUM_CORES + lax.axis_index("c")
        base = wid * b_per_w
        pltpu.sync_copy(idx_hbm.at[pl.ds(base, b_per_w)], idx_v)
        out_flat = out_hbm

        def start_gather(g, b):
            pltpu.async_copy(
                table_hbm.at[idx_v.at[pl.ds(g * _CHUNK, _CHUNK)]],
                rows_v.at[b],
                gsem.at[b],
            )

        def start_store(g, b):
            pltpu.async_copy(
                rows_v.at[b],
                out_flat.at[pl.ds(base + g * _CHUNK, _CHUNK)],
                ssem.at[b],
            )

        def wait_gather(g, b):
            pltpu.make_async_copy(
                table_hbm.at[idx_v.at[pl.ds(g * _CHUNK, _CHUNK)]],
                rows_v.at[b],
                gsem.at[b],
            ).wait()

        def wait_store(g, b):
            pltpu.make_async_copy(
                rows_v.at[b],
                out_flat.at[pl.ds(base + g * _CHUNK, _CHUNK)],
                ssem.at[b],
            ).wait()

        for i in range(_NBUF):
            start_gather(i, i)
            if i == _NBUF - 1:
                wait_gather(0, 0)
                start_store(0, 0)

        @pl.loop(_NBUF, nchunks)
        def _(i):
            b = lax.rem(i, _NBUF)
            bp = lax.rem(i + 1, _NBUF)
            wait_store(i - _NBUF, b)
            start_gather(i, b)
            wait_gather(i - _NBUF + 1, bp)
            start_store(i - _NBUF + 1, bp)

        for i in range(nchunks, nchunks + _NBUF):
            b = i % _NBUF
            bp = (i + 1) % _NBUF
            wait_store(i - _NBUF, b)
            if i < nchunks + _NBUF - 1:
                wait_gather(i - _NBUF + 1, bp)
                start_store(i - _NBUF + 1, bp)

    return gather


def kernel(indices, weight):
    num_rows, dim = weight.shape
    nbatch, nfields = indices.shape
    flat_t = indices.T.reshape(-1)
    wpad = jnp.pad(weight, ((0, 0), (0, dim)))
    gather = _make_gather(num_rows, dim, flat_t.shape[0], nfields)
    out = gather(wpad, flat_t)
    out = out.reshape(nfields, nbatch, 2 * dim)[:, :, :dim]
    return out.transpose(1, 0, 2)
